# Initial kernel scaffold; baseline (speedup 1.0000x reference)
import functools

import jax
import jax.numpy as jnp
from jax import lax
from jax.experimental import pallas as pl
from jax.experimental.pallas import tpu as pltpu
from jax.experimental.pallas import tpu_sc as plsc

T, B, L, V, D = 26, 4096, 20, 100000, 16
NC, NS = 2, 16
NW = NC * NS
NB = B // NW
NBAGS = T * NB
TRASH = NBAGS
LANES = 16


def _emb_body(idx_hbm, len_hbm, tab_hbm, out_hbm,
              idxraw, idxg, dstv, rows, acc, lens, gsem, ssem):
    wid = lax.axis_index("s") * NC + lax.axis_index("c")
    b0 = wid * NB
    iota = lax.iota(jnp.int32, LANES)
    iotaT = iota * T
    zero16 = jnp.zeros((LANES,), jnp.float32)

    def zbody(i, carry):
        acc[i, :] = zero16
        return carry

    lax.fori_loop(0, NBAGS + 1, zbody, 0)

    pltpu.sync_copy(len_hbm.at[:, pl.ds(b0, NB)], lens)

    def tbody(t, carry):
        pltpu.sync_copy(idx_hbm.at[t, pl.ds(b0, NB), :], idxraw)
        toff = t * V

        def lbody(l, c2):
            lvec = jnp.full((LANES,), l, jnp.int32)
            for c in range(NB // LANES):
                rowsel = c * LANES + iota
                raw = plsc.load_gather(idxraw, [rowsel, lvec])
                idxg[l, pl.ds(c * LANES, LANES)] = raw + toff
                lensrow = lens[t, pl.ds(c * LANES, LANES)]
                dst = jnp.where(lensrow > l, iotaT + (c * LANES * T + t),
                                TRASH)
                dstv[l, pl.ds(c * LANES, LANES)] = dst
            return c2

        lax.fori_loop(0, L, lbody, 0)

        gcopies = [
            pltpu.async_copy(tab_hbm.at[idxg.at[l]],
                             rows.at[pl.ds(l * NB, NB)], gsem)
            for l in range(L)
        ]
        for cp in gcopies:
            cp.wait()
        scopies = [
            pltpu.async_copy(rows.at[pl.ds(l * NB, NB)],
                             acc.at[dstv.at[l]], ssem, add=True)
            for l in range(L)
        ]
        for cp in scopies:
            cp.wait()
        return carry

    lax.fori_loop(0, T, tbody, 0)

    tvec = jnp.full((LANES,), T, jnp.int32)

    def gbody(g, carry):
        nvec = g * LANES + iota
        tt = lax.rem(nvec, tvec)
        bi = lax.div(nvec, tvec)
        l16 = plsc.load_gather(lens, [tt, bi])
        inv = 1.0 / jnp.maximum(l16, 1).astype(jnp.float32)
        for d in range(D):
            dvec = jnp.full((LANES,), d, jnp.int32)
            v = plsc.load_gather(acc, [nvec, dvec])
            plsc.store_scatter(acc, [nvec, dvec], v * inv)
        return carry

    lax.fori_loop(0, NBAGS // LANES, gbody, 0)

    pltpu.sync_copy(acc.at[pl.ds(0, NBAGS)],
                    out_hbm.at[pl.ds(wid * NBAGS, NBAGS)])


_emb = functools.partial(
    pl.kernel,
    out_type=jax.ShapeDtypeStruct((B * T, D), jnp.float32),
    mesh=plsc.VectorSubcoreMesh(core_axis_name="c", subcore_axis_name="s",
                                num_cores=NC, num_subcores=NS),
    scratch_types=[
        pltpu.VMEM((NB, L), jnp.int32),
        pltpu.VMEM((L, NB), jnp.int32),
        pltpu.VMEM((L, NB), jnp.int32),
        pltpu.VMEM((L * NB, D), jnp.float32),
        pltpu.VMEM((NBAGS + 1, D), jnp.float32),
        pltpu.VMEM((T, NB), jnp.int32),
        pltpu.SemaphoreType.DMA,
        pltpu.SemaphoreType.DMA,
    ],
)(_emb_body)


def kernel(indices, lengths, tables):
    tab2d = tables.reshape(T * V, D)
    out2d = _emb(indices, lengths, tab2d)
    return out2d.reshape(B, T * D)


# speedup vs baseline: 4.0397x; 4.0397x over previous
import functools

import jax
import jax.numpy as jnp
from jax import lax
from jax.experimental import pallas as pl
from jax.experimental.pallas import tpu as pltpu
from jax.experimental.pallas import tpu_sc as plsc

T, B, L, V, D = 26, 4096, 20, 100000, 16
NC, NS = 2, 16
NW = NC * NS
NB = B // NW
NBAGS = T * NB
TRASH = NBAGS
LANES = 16
NROWS = L * NB


def _emb_body(idx_hbm, len_hbm, tab_hbm, out_hbm,
              idxraw0, idxraw1, idxg0, idxg1, dstv0, dstv1,
              rows0, rows1, accs, lens, gsem, ssem, isem):
    sid = lax.axis_index("s")
    wid = sid * NC + lax.axis_index("c")
    b0 = wid * NB
    iota = lax.iota(jnp.int32, LANES)
    iotaT = iota * T
    zero16 = jnp.zeros((LANES,), jnp.float32)
    acc = accs.at[sid]

    HROWS = L // 2 * NB

    def zbody(i, carry):
        rows0[i, :] = zero16
        return carry

    lax.fori_loop(0, HROWS, zbody, 0)
    pltpu.sync_copy(rows0, acc.at[pl.ds(0, HROWS)])
    pltpu.sync_copy(rows0, acc.at[pl.ds(HROWS, HROWS)])
    pltpu.sync_copy(rows0.at[pl.ds(0, NBAGS + 1 - 2 * HROWS)],
                    acc.at[pl.ds(2 * HROWS, NBAGS + 1 - 2 * HROWS)])

    pltpu.sync_copy(len_hbm.at[:, pl.ds(b0, NB)], lens)

    def start_idx(t, raw):
        return pltpu.async_copy(idx_hbm.at[t, pl.ds(b0, NB), :], raw, isem)

    def build(t, raw, gidx, gdst):
        toff = t * V

        def lbody(l, carry):
            lvec = jnp.full((LANES,), l, jnp.int32)
            for c in range(NB // LANES):
                rowsel = c * LANES + iota
                rawv = plsc.load_gather(raw, [rowsel, lvec])
                gidx[l, pl.ds(c * LANES, LANES)] = rawv + toff
                lensrow = lens[t, pl.ds(c * LANES, LANES)]
                gdst[l, pl.ds(c * LANES, LANES)] = jnp.where(
                    lensrow > l, iotaT + (c * LANES * T + t), TRASH)
            return carry

        lax.fori_loop(0, L, lbody, 0)

    LH = L // 2

    def fire_gathers(gidx, j0, rows):
        for j in range(LH):
            pltpu.async_copy(tab_hbm.at[gidx.at[j0 + j]],
                             rows.at[pl.ds(j * NB, NB)], gsem)

    def drain_gathers(gidx, j0, rows):
        for j in range(LH):
            pltpu.make_async_copy(tab_hbm.at[gidx.at[j0 + j]],
                                  rows.at[pl.ds(j * NB, NB)], gsem).wait()

    def scatter_adds(rows, gdst, j0):
        cps = [pltpu.async_copy(rows.at[pl.ds(j * NB, NB)],
                                acc.at[gdst.at[j0 + j]], ssem, add=True)
               for j in range(LH)]
        for cp in cps:
            cp.wait()

    idxbufs = ((idxraw0, idxg0, dstv0), (idxraw1, idxg1, dstv1))
    rowbufs = (rows0, rows1)
    tmax = T - 1

    start_idx(0, idxraw0)
    pltpu.make_async_copy(idx_hbm.at[0, pl.ds(b0, NB), :], idxraw0,
                          isem).wait()
    build(0, idxraw0, idxg0, dstv0)
    start_idx(1, idxraw1)
    fire_gathers(idxg0, 0, rows0)

    def sbody(s, carry):
        for q in range(4):
            t = 2 * s + (q // 2)
            iset = q // 2
            half = q % 2
            rawc, gidxc, gdstc = idxbufs[iset]
            rowsc = rowbufs[half]
            rowsn = rowbufs[1 - half]
            if half == 0:
                tn = jnp.minimum(t + 1, tmax)
                rawn, gidxn, gdstn = idxbufs[1 - iset]
                pltpu.make_async_copy(idx_hbm.at[tn, pl.ds(b0, NB), :],
                                      rawn, isem).wait()
                build(tn, rawn, gidxn, gdstn)
                start_idx(jnp.minimum(t + 2, tmax), rawc)
                gidx_next, j0_next = gidxc, LH
            else:
                gidx_next, j0_next = idxbufs[1 - iset][1], 0
            drain_gathers(gidxc, half * LH, rowsc)
            fire_gathers(gidx_next, j0_next, rowsn)
            scatter_adds(rowsc, gdstc, half * LH)
        return carry

    lax.fori_loop(0, T // 2, sbody, 0)

    pltpu.make_async_copy(idx_hbm.at[tmax, pl.ds(b0, NB), :], idxraw0,
                          isem).wait()
    drain_gathers(idxg0, 0, rows0)

    tvec = jnp.full((LANES,), T, jnp.int32)

    for off, n in ((0, HROWS), (HROWS, HROWS), (2 * HROWS, NBAGS - 2 * HROWS)):
        pltpu.sync_copy(acc.at[pl.ds(off, n)], rows0.at[pl.ds(0, n)])

        def gbody(g, carry):
            loc = g * LANES + iota
            nvec = off + loc
            tt = lax.rem(nvec, tvec)
            bi = lax.div(nvec, tvec)
            l16 = plsc.load_gather(lens, [tt, bi])
            inv = 1.0 / jnp.maximum(l16, 1).astype(jnp.float32)
            for d in range(D):
                dvec = jnp.full((LANES,), d, jnp.int32)
                v = plsc.load_gather(rows0, [loc, dvec])
                plsc.store_scatter(rows0, [loc, dvec], v * inv)
            return carry

        lax.fori_loop(0, n // LANES, gbody, 0)
        pltpu.sync_copy(rows0.at[pl.ds(0, n)],
                        out_hbm.at[pl.ds(wid * NBAGS + off, n)])


_emb = functools.partial(
    pl.kernel,
    compiler_params=pltpu.CompilerParams(
        needs_layout_passes=False, use_tc_tiling_on_sc=False),
    out_type=jax.ShapeDtypeStruct((B * T, D), jnp.float32),
    mesh=plsc.VectorSubcoreMesh(core_axis_name="c", subcore_axis_name="s",
                                num_cores=NC, num_subcores=NS),
    scratch_types=[
        pltpu.VMEM((NB, L), jnp.int32),
        pltpu.VMEM((NB, L), jnp.int32),
        pltpu.VMEM((L, NB), jnp.int32),
        pltpu.VMEM((L, NB), jnp.int32),
        pltpu.VMEM((L, NB), jnp.int32),
        pltpu.VMEM((L, NB), jnp.int32),
        pltpu.VMEM((L // 2 * NB, D), jnp.float32),
        pltpu.VMEM((L // 2 * NB, D), jnp.float32),
        pltpu.VMEM_SHARED((NS, NBAGS + 1, D), jnp.float32),
        pltpu.VMEM((T, NB), jnp.int32),
        pltpu.SemaphoreType.DMA,
        pltpu.SemaphoreType.DMA,
        pltpu.SemaphoreType.DMA,
    ],
)(_emb_body)


def kernel(indices, lengths, tables):
    tab2d = tables.reshape(T * V, D)
    out2d = _emb(indices, lengths, tab2d)
    return out2d.reshape(B, T * D)
